# baseline (speedup 1.0000x reference)
import jax
import jax.numpy as jnp
from jax.experimental import pallas as pl
from jax.experimental.pallas import tpu as pltpu

_N, _D = 16384, 1000
_R = 256
_G = _N // _R


def _ramp_body(tgt_ref, inp_ref, out_ref):
    g = pl.program_id(0)
    x = inp_ref[...]
    t = tgt_ref[0, 0, :]
    col = jax.lax.broadcasted_iota(jnp.int32, (_R, _D), 1)
    is_t = col == t[:, None]
    v_y = jnp.sum(jnp.where(is_t, x, 0.0), axis=1)
    m_neq = jnp.max(jnp.where(is_t, -jnp.inf, x), axis=1)
    r = m_neq - v_y
    loss = jnp.clip(1.0 + r, 0.0, 1.0)

    @pl.when(g == 0)
    def _():
        out_ref[...] = jnp.zeros((1, 1), jnp.float32)

    out_ref[...] += jnp.sum(loss).reshape(1, 1)


def kernel(inp, tgt):
    tgt3 = tgt.astype(jnp.int32).reshape(_G, 1, _R)
    acc = pl.pallas_call(
        _ramp_body,
        grid=(_G,),
        in_specs=[
            pl.BlockSpec((1, 1, _R), lambda i: (i, 0, 0)),
            pl.BlockSpec((_R, _D), lambda i: (i, 0)),
        ],
        out_specs=pl.BlockSpec((1, 1), lambda i: (0, 0)),
        out_shape=jax.ShapeDtypeStruct((1, 1), jnp.float32),
        compiler_params=pltpu.CompilerParams(
            dimension_semantics=("arbitrary",),
        ),
    )(tgt3, inp)
    return (acc[0] / _N).astype(jnp.float32)


# speedup vs baseline: 1.0020x; 1.0020x over previous
import jax
import jax.numpy as jnp
from jax.experimental import pallas as pl
from jax.experimental.pallas import tpu as pltpu

_N, _D = 16384, 1000
_R = 256
_G = _N // _R


def _ramp_body(tgt_ref, inp_ref, out_ref):
    x = inp_ref[...]
    t = tgt_ref[0, 0, :]
    col = jax.lax.broadcasted_iota(jnp.int32, (_R, _D), 1)
    is_t = col == t[:, None]
    v_y = jnp.sum(jnp.where(is_t, x, 0.0), axis=1)
    m_neq = jnp.max(jnp.where(is_t, -jnp.inf, x), axis=1)
    r = m_neq - v_y
    loss = jnp.clip(1.0 + r, 0.0, 1.0)
    out_ref[...] = jnp.sum(loss).reshape(1, 1, 1)


def kernel(inp, tgt):
    tgt3 = tgt.astype(jnp.int32).reshape(_G, 1, _R)
    partials = pl.pallas_call(
        _ramp_body,
        grid=(_G,),
        in_specs=[
            pl.BlockSpec((1, 1, _R), lambda i: (i, 0, 0)),
            pl.BlockSpec((_R, _D), lambda i: (i, 0)),
        ],
        out_specs=pl.BlockSpec((1, 1, 1), lambda i: (i, 0, 0)),
        out_shape=jax.ShapeDtypeStruct((_G, 1, 1), jnp.float32),
        compiler_params=pltpu.CompilerParams(
            dimension_semantics=("parallel",),
        ),
    )(tgt3, inp)
    return (jnp.sum(partials) / _N).reshape(1)


# speedup vs baseline: 1.1786x; 1.1763x over previous
import jax
import jax.numpy as jnp
from jax.experimental import pallas as pl
from jax.experimental.pallas import tpu as pltpu

_N, _D = 16384, 1000
_R = 512
_G = _N // _R


def _ramp_body(tgt_ref, inp_ref, out_ref):
    x = inp_ref[...]
    t = tgt_ref[0, 0, :]
    col = jax.lax.broadcasted_iota(jnp.int32, (_R, _D), 1)
    is_t = col == t[:, None]
    v_y = jnp.sum(jnp.where(is_t, x, 0.0), axis=1)
    m_neq = jnp.max(jnp.where(is_t, -jnp.inf, x), axis=1)
    r = m_neq - v_y
    loss = jnp.clip(1.0 + r, 0.0, 1.0)
    out_ref[...] = jnp.sum(loss).reshape(1, 1, 1)


def kernel(inp, tgt):
    tgt3 = tgt.astype(jnp.int32).reshape(_G, 1, _R)
    partials = pl.pallas_call(
        _ramp_body,
        grid=(_G,),
        in_specs=[
            pl.BlockSpec((1, 1, _R), lambda i: (i, 0, 0)),
            pl.BlockSpec((_R, _D), lambda i: (i, 0)),
        ],
        out_specs=pl.BlockSpec((1, 1, 1), lambda i: (i, 0, 0)),
        out_shape=jax.ShapeDtypeStruct((_G, 1, 1), jnp.float32),
        compiler_params=pltpu.CompilerParams(
            dimension_semantics=("parallel",),
        ),
    )(tgt3, inp)
    return (jnp.sum(partials) / _N).reshape(1)


# speedup vs baseline: 1.3117x; 1.1129x over previous
import jax
import jax.numpy as jnp
from jax.experimental import pallas as pl
from jax.experimental.pallas import tpu as pltpu

_N, _D = 16384, 1000
_R = 1024
_G = _N // _R


def _ramp_body(tgt_ref, inp_ref, out_ref):
    x = inp_ref[...]
    t = tgt_ref[0, 0, :]
    col = jax.lax.broadcasted_iota(jnp.int32, (_R, _D), 1)
    is_t = col == t[:, None]
    v_y = jnp.sum(jnp.where(is_t, x, 0.0), axis=1)
    m_neq = jnp.max(jnp.where(is_t, -jnp.inf, x), axis=1)
    r = m_neq - v_y
    loss = jnp.clip(1.0 + r, 0.0, 1.0)
    out_ref[...] = jnp.sum(loss).reshape(1, 1, 1)


def kernel(inp, tgt):
    tgt3 = tgt.astype(jnp.int32).reshape(_G, 1, _R)
    partials = pl.pallas_call(
        _ramp_body,
        grid=(_G,),
        in_specs=[
            pl.BlockSpec((1, 1, _R), lambda i: (i, 0, 0)),
            pl.BlockSpec((_R, _D), lambda i: (i, 0)),
        ],
        out_specs=pl.BlockSpec((1, 1, 1), lambda i: (i, 0, 0)),
        out_shape=jax.ShapeDtypeStruct((_G, 1, 1), jnp.float32),
        compiler_params=pltpu.CompilerParams(
            dimension_semantics=("parallel",),
        ),
    )(tgt3, inp)
    return (jnp.sum(partials) / _N).reshape(1)
